# baseline (speedup 1.0000x reference)
import functools

import jax
import jax.numpy as jnp
from jax import lax
from jax.experimental import pallas as pl
from jax.experimental.pallas import tpu as pltpu
from jax.experimental.pallas import tpu_sc as plsc

_NC = 2
_NS = 16
_LANES = 16
_K = 112


def _sc_partials(src_r, dst_r, wb_r, xi32, n, d, chunks):
    n_acc = ((n + _NS * 128 - 1) // (_NS * 128)) * (_NS * 128)
    rows_per_tile = n_acc // _NS
    grp = _K // _LANES
    mesh = plsc.VectorSubcoreMesh(core_axis_name="c", subcore_axis_name="s")

    @functools.partial(
        pl.kernel,
        out_type=jax.ShapeDtypeStruct((_NC, n_acc, d), jnp.float32),
        mesh=mesh,
        compiler_params=pltpu.CompilerParams(use_tc_tiling_on_sc=False),
        scratch_types=[
            pltpu.VMEM((3, _K), jnp.int32),
            pltpu.VMEM((3, _K), jnp.int32),
            pltpu.VMEM((_K, d // 2), jnp.int32),
            pltpu.VMEM((_K, d // 2), jnp.int32),
            pltpu.VMEM((_K, d), jnp.float32),
            pltpu.VMEM((_K, d), jnp.float32),
            pltpu.VMEM((_K,), jnp.int32),
            pltpu.VMEM((_K,), jnp.int32),
            pltpu.VMEM_SHARED((n_acc, d), jnp.float32),
            pltpu.SemaphoreType.DMA,
            pltpu.SemaphoreType.DMA,
            pltpu.SemaphoreType.DMA,
            pltpu.SemaphoreType.DMA,
            pltpu.SemaphoreType.DMA,
            pltpu.SemaphoreType.DMA,
        ],
    )
    def k(src_hbm, dst_hbm, wb_hbm, x_hbm, out_hbm, ib0, ib1, rbf0, rbf1,
          rf0, rf1, dv0, dv1, acc, isem0, isem1, gsem0, gsem1, ssem0, ssem1):
        cid = lax.axis_index("c")
        sid = lax.axis_index("s")
        wid = cid * _NS + sid

        ib = (ib0, ib1)
        isem = (isem0, isem1)
        rbf = (rbf0, rbf1)
        gsem = (gsem0, gsem1)
        rf = (rf0, rf1)
        dv = (dv0, dv1)
        ssem = (ssem0, ssem1)

        zeros16 = jnp.zeros((_LANES,), jnp.float32)

        def zrow(r, carry):
            for j in range(d // _LANES):
                rf0[r, pl.ds(j * _LANES, _LANES)] = zeros16
            return carry

        lax.fori_loop(0, _K, zrow, 0)
        row0 = sid * rows_per_tile
        nfull = rows_per_tile // _K
        for i in range(nfull):
            pltpu.sync_copy(rf0, acc.at[pl.ds(row0 + i * _K, _K)])
        rem = rows_per_tile - nfull * _K
        if rem:
            pltpu.sync_copy(rf0.at[pl.ds(0, rem)],
                            acc.at[pl.ds(row0 + nfull * _K, rem)])
        plsc.subcore_barrier()

        def load_meta(b, c, copy_fn):
            copy_fn(src_hbm.at[wid, c], ib[b].at[0], isem[b])
            copy_fn(dst_hbm.at[wid, c], ib[b].at[1], isem[b])
            copy_fn(wb_hbm.at[wid, c], ib[b].at[2], isem[b])

        def phase(b, c):
            @pl.when(c + 1 < chunks)
            def _():
                load_meta(b ^ 1, 0,
                          lambda s_, d_, m_: pltpu.make_async_copy(
                              s_, d_, m_).wait())
                pltpu.async_copy(
                    x_hbm.at[ib[b ^ 1].at[0]], rbf[b ^ 1], gsem[b ^ 1])

            pltpu.make_async_copy(
                x_hbm.at[ib[b].at[0]], rbf[b], gsem[b]).wait()

            @pl.when(c >= 2)
            def _():
                pltpu.make_async_copy(
                    rf[b], acc.at[dv[b]], ssem[b]).wait()

            @plsc.parallel_loop(0, grp, unroll=2)
            def _scale(g):
                wvec = lax.bitcast_convert_type(
                    ib[b][2, pl.ds(g * _LANES, _LANES)], jnp.float32)
                for i in range(_LANES):
                    ws = wvec[i]
                    eb = g * _LANES + i
                    for j in range(d // (2 * _LANES)):
                        v = rbf[b][eb, pl.ds(j * _LANES, _LANES)]
                        lo = lax.bitcast_convert_type(v << 16, jnp.float32)
                        hi = lax.bitcast_convert_type(
                            v & jnp.int32(-65536), jnp.float32)
                        base = j * 2 * _LANES
                        rf[b][eb, pl.ds(base, _LANES)] = lo * ws
                        rf[b][eb, pl.ds(base + _LANES, _LANES)] = hi * ws

            for j in range(grp):
                dv[b][pl.ds(j * _LANES, _LANES)] = (
                    ib[b][1, pl.ds(j * _LANES, _LANES)])
            pltpu.async_copy(rf[b], acc.at[dv[b]], ssem[b], add=True)

            @pl.when(c + 2 < chunks)
            def _():
                load_meta(b, c + 2,
                          lambda s_, d_, m_: pltpu.async_copy(s_, d_, m_))

        load_meta(0, 0, lambda s_, d_, m_: pltpu.sync_copy(s_, d_))
        pltpu.async_copy(x_hbm.at[ib0.at[0]], rbf0, gsem0)
        load_meta(1, 1, lambda s_, d_, m_: pltpu.async_copy(s_, d_, m_))

        def pair_body(it, carry):
            phase(0, 2 * it)
            phase(1, 2 * it + 1)
            return carry

        lax.fori_loop(0, chunks // 2, pair_body, 0)

        pltpu.make_async_copy(rf0, acc.at[dv0], ssem0).wait()
        pltpu.make_async_copy(rf1, acc.at[dv1], ssem1).wait()
        plsc.subcore_barrier()
        pltpu.sync_copy(
            acc.at[pl.ds(row0, rows_per_tile)],
            out_hbm.at[cid, pl.ds(row0, rows_per_tile)])

    return k(src_r, dst_r, wb_r, xi32)


def _mix(x, parts, alpha):
    n, d = x.shape
    blk = 1000

    def body(a_ref, x_ref, p_ref, o_ref):
        a = a_ref[0]
        o_ref[...] = a * x_ref[...] + (1.0 - a) * (p_ref[0] + p_ref[1])

    return pl.pallas_call(
        body,
        grid=(n // blk,),
        in_specs=[
            pl.BlockSpec(memory_space=pltpu.SMEM),
            pl.BlockSpec((blk, d), lambda i: (i, 0)),
            pl.BlockSpec((2, blk, d), lambda i: (0, i, 0)),
        ],
        out_specs=pl.BlockSpec((blk, d), lambda i: (i, 0)),
        out_shape=jax.ShapeDtypeStruct((n, d), jnp.float32),
    )(alpha, x, parts)


def kernel(x, edge_index, edge_weight, alpha):
    n, d = x.shape
    e = edge_weight.shape[0]
    n_workers = _NC * _NS
    per = n_workers * _K * 2
    e_pad = ((e + per - 1) // per) * per
    pad = e_pad - e
    src = edge_index[1].astype(jnp.int32)
    dst = edge_index[0].astype(jnp.int32)
    w = edge_weight.astype(jnp.float32)
    if pad:
        src = jnp.concatenate([src, jnp.zeros((pad,), jnp.int32)])
        dst = jnp.concatenate([dst, jnp.zeros((pad,), jnp.int32)])
        w = jnp.concatenate([w, jnp.zeros((pad,), jnp.float32)])
    chunks = e_pad // (n_workers * _K)
    wbits = lax.bitcast_convert_type(w, jnp.int32)
    src_r = src.reshape(n_workers, chunks, _K)
    dst_r = dst.reshape(n_workers, chunks, _K)
    wb_r = wbits.reshape(n_workers, chunks, _K)
    xbf = (x.astype(jnp.bfloat16)
           .reshape(n, d // 32, 2, _LANES).swapaxes(-1, -2)
           .reshape(n, d // 2, 2))
    xi32 = lax.bitcast_convert_type(xbf, jnp.int32)
    parts = _sc_partials(src_r, dst_r, wb_r, xi32, n, d, chunks)
    return _mix(x, parts, alpha.astype(jnp.float32))


# speedup vs baseline: 1.0002x; 1.0002x over previous
import functools

import jax
import jax.numpy as jnp
from jax import lax
from jax.experimental import pallas as pl
from jax.experimental.pallas import tpu as pltpu
from jax.experimental.pallas import tpu_sc as plsc

_NC = 2
_NS = 16
_LANES = 16
_K = 112


def _sc_partials(src_r, dst_r, wb_r, xi32, n, d, chunks):
    n_acc = ((n + _NS * 128 - 1) // (_NS * 128)) * (_NS * 128)
    rows_per_tile = n_acc // _NS
    grp = _K // _LANES
    mesh = plsc.VectorSubcoreMesh(core_axis_name="c", subcore_axis_name="s")

    @functools.partial(
        pl.kernel,
        out_type=jax.ShapeDtypeStruct((_NC, n_acc, d), jnp.float32),
        mesh=mesh,
        compiler_params=pltpu.CompilerParams(use_tc_tiling_on_sc=False),
        scratch_types=[
            pltpu.VMEM((3, _K), jnp.int32),
            pltpu.VMEM((3, _K), jnp.int32),
            pltpu.VMEM((_K, d // 2), jnp.int32),
            pltpu.VMEM((_K, d // 2), jnp.int32),
            pltpu.VMEM((_K, d), jnp.float32),
            pltpu.VMEM((_K, d), jnp.float32),
            pltpu.VMEM((_K,), jnp.int32),
            pltpu.VMEM((_K,), jnp.int32),
            pltpu.VMEM_SHARED((n_acc, d), jnp.float32),
            pltpu.SemaphoreType.DMA,
            pltpu.SemaphoreType.DMA,
            pltpu.SemaphoreType.DMA,
            pltpu.SemaphoreType.DMA,
            pltpu.SemaphoreType.DMA,
            pltpu.SemaphoreType.DMA,
        ],
    )
    def k(src_hbm, dst_hbm, wb_hbm, x_hbm, out_hbm, ib0, ib1, rbf0, rbf1,
          rf0, rf1, dv0, dv1, acc, isem0, isem1, gsem0, gsem1, ssem0, ssem1):
        cid = lax.axis_index("c")
        sid = lax.axis_index("s")
        wid = cid * _NS + sid

        ib = (ib0, ib1)
        isem = (isem0, isem1)
        rbf = (rbf0, rbf1)
        gsem = (gsem0, gsem1)
        rf = (rf0, rf1)
        dv = (dv0, dv1)
        ssem = (ssem0, ssem1)

        zeros16 = jnp.zeros((_LANES,), jnp.float32)

        def zrow(r, carry):
            for j in range(d // _LANES):
                rf0[r, pl.ds(j * _LANES, _LANES)] = zeros16
            return carry

        lax.fori_loop(0, _K, zrow, 0)
        row0 = sid * rows_per_tile
        nfull = rows_per_tile // _K
        for i in range(nfull):
            pltpu.sync_copy(rf0, acc.at[pl.ds(row0 + i * _K, _K)])
        rem = rows_per_tile - nfull * _K
        if rem:
            pltpu.sync_copy(rf0.at[pl.ds(0, rem)],
                            acc.at[pl.ds(row0 + nfull * _K, rem)])
        plsc.subcore_barrier()

        def load_meta(b, c, copy_fn):
            copy_fn(src_hbm.at[wid, c], ib[b].at[0], isem[b])
            copy_fn(dst_hbm.at[wid, c], ib[b].at[1], isem[b])
            copy_fn(wb_hbm.at[wid, c], ib[b].at[2], isem[b])

        def phase(b, c):
            @pl.when(c + 1 < chunks)
            def _():
                load_meta(b ^ 1, 0,
                          lambda s_, d_, m_: pltpu.make_async_copy(
                              s_, d_, m_).wait())
                h = _K // 2
                pltpu.async_copy(
                    x_hbm.at[ib[b ^ 1].at[0, pl.ds(0, h)]],
                    rbf[b ^ 1].at[pl.ds(0, h)], gsem[b ^ 1])
                pltpu.async_copy(
                    x_hbm.at[ib[b ^ 1].at[0, pl.ds(h, h)]],
                    rbf[b ^ 1].at[pl.ds(h, h)], gsem[b ^ 1])

            h = _K // 2
            pltpu.make_async_copy(
                x_hbm.at[ib[b].at[0, pl.ds(0, h)]],
                rbf[b].at[pl.ds(0, h)], gsem[b]).wait()
            pltpu.make_async_copy(
                x_hbm.at[ib[b].at[0, pl.ds(h, h)]],
                rbf[b].at[pl.ds(h, h)], gsem[b]).wait()

            @pl.when(c >= 2)
            def _():
                pltpu.make_async_copy(
                    rf[b], acc.at[dv[b]], ssem[b]).wait()

            @plsc.parallel_loop(0, grp, unroll=2)
            def _scale(g):
                wvec = lax.bitcast_convert_type(
                    ib[b][2, pl.ds(g * _LANES, _LANES)], jnp.float32)
                for i in range(_LANES):
                    ws = wvec[i]
                    eb = g * _LANES + i
                    for j in range(d // (2 * _LANES)):
                        v = rbf[b][eb, pl.ds(j * _LANES, _LANES)]
                        lo = lax.bitcast_convert_type(v << 16, jnp.float32)
                        hi = lax.bitcast_convert_type(
                            v & jnp.int32(-65536), jnp.float32)
                        base = j * 2 * _LANES
                        rf[b][eb, pl.ds(base, _LANES)] = lo * ws
                        rf[b][eb, pl.ds(base + _LANES, _LANES)] = hi * ws

            for j in range(grp):
                dv[b][pl.ds(j * _LANES, _LANES)] = (
                    ib[b][1, pl.ds(j * _LANES, _LANES)])
            pltpu.async_copy(rf[b], acc.at[dv[b]], ssem[b], add=True)

            @pl.when(c + 2 < chunks)
            def _():
                load_meta(b, c + 2,
                          lambda s_, d_, m_: pltpu.async_copy(s_, d_, m_))

        load_meta(0, 0, lambda s_, d_, m_: pltpu.sync_copy(s_, d_))
        h0 = _K // 2
        pltpu.async_copy(x_hbm.at[ib0.at[0, pl.ds(0, h0)]],
                         rbf0.at[pl.ds(0, h0)], gsem0)
        pltpu.async_copy(x_hbm.at[ib0.at[0, pl.ds(h0, h0)]],
                         rbf0.at[pl.ds(h0, h0)], gsem0)
        load_meta(1, 1, lambda s_, d_, m_: pltpu.async_copy(s_, d_, m_))

        def pair_body(it, carry):
            phase(0, 2 * it)
            phase(1, 2 * it + 1)
            return carry

        lax.fori_loop(0, chunks // 2, pair_body, 0)

        pltpu.make_async_copy(rf0, acc.at[dv0], ssem0).wait()
        pltpu.make_async_copy(rf1, acc.at[dv1], ssem1).wait()
        plsc.subcore_barrier()
        pltpu.sync_copy(
            acc.at[pl.ds(row0, rows_per_tile)],
            out_hbm.at[cid, pl.ds(row0, rows_per_tile)])

    return k(src_r, dst_r, wb_r, xi32)


def _mix(x, parts, alpha):
    n, d = x.shape
    blk = 1000

    def body(a_ref, x_ref, p_ref, o_ref):
        a = a_ref[0]
        o_ref[...] = a * x_ref[...] + (1.0 - a) * (p_ref[0] + p_ref[1])

    return pl.pallas_call(
        body,
        grid=(n // blk,),
        in_specs=[
            pl.BlockSpec(memory_space=pltpu.SMEM),
            pl.BlockSpec((blk, d), lambda i: (i, 0)),
            pl.BlockSpec((2, blk, d), lambda i: (0, i, 0)),
        ],
        out_specs=pl.BlockSpec((blk, d), lambda i: (i, 0)),
        out_shape=jax.ShapeDtypeStruct((n, d), jnp.float32),
    )(alpha, x, parts)


def kernel(x, edge_index, edge_weight, alpha):
    n, d = x.shape
    e = edge_weight.shape[0]
    n_workers = _NC * _NS
    per = n_workers * _K * 2
    e_pad = ((e + per - 1) // per) * per
    pad = e_pad - e
    src = edge_index[1].astype(jnp.int32)
    dst = edge_index[0].astype(jnp.int32)
    w = edge_weight.astype(jnp.float32)
    if pad:
        src = jnp.concatenate([src, jnp.zeros((pad,), jnp.int32)])
        dst = jnp.concatenate([dst, jnp.zeros((pad,), jnp.int32)])
        w = jnp.concatenate([w, jnp.zeros((pad,), jnp.float32)])
    chunks = e_pad // (n_workers * _K)
    wbits = lax.bitcast_convert_type(w, jnp.int32)
    src_r = src.reshape(n_workers, chunks, _K)
    dst_r = dst.reshape(n_workers, chunks, _K)
    wb_r = wbits.reshape(n_workers, chunks, _K)
    xbf = (x.astype(jnp.bfloat16)
           .reshape(n, d // 32, 2, _LANES).swapaxes(-1, -2)
           .reshape(n, d // 2, 2))
    xi32 = lax.bitcast_convert_type(xbf, jnp.int32)
    parts = _sc_partials(src_r, dst_r, wb_r, xi32, n, d, chunks)
    return _mix(x, parts, alpha.astype(jnp.float32))
